# baseline (speedup 1.0000x reference)
import functools

import jax
import jax.numpy as jnp
from jax import lax
from jax.experimental import pallas as pl
from jax.experimental.pallas import tpu as pltpu
from jax.experimental.pallas import tpu_sc as plsc


def kernel(inputs, pos_embedding):
    B, S, D = inputs.shape
    x = inputs.reshape(B, S * D)
    p = pos_embedding.reshape(S * D)

    info = plsc.get_sparse_core_info()
    NC, NS, L = info.num_cores, info.num_subcores, info.num_lanes
    NW = NC * NS
    rows_w = S // NW
    CH = 8
    n_chunks = rows_w // CH
    CW = CH * D

    mesh = plsc.VectorSubcoreMesh(core_axis_name="c", subcore_axis_name="s")

    @functools.partial(
        pl.kernel,
        mesh=mesh,
        out_type=jax.ShapeDtypeStruct((B, S * D), jnp.float32),
        scratch_types=[
            pltpu.VMEM((CW,), jnp.float32),
            pltpu.VMEM((B, CW), jnp.float32),
        ],
    )
    def k(x_hbm, p_hbm, o_hbm, p_v, x_v):
        wid = lax.axis_index("s") * NC + lax.axis_index("c")
        base = wid * (rows_w * D)

        def chunk(ci, carry):
            off = base + ci * CW
            pltpu.sync_copy(p_hbm.at[pl.ds(off, CW)], p_v)
            for b in range(B):
                pltpu.sync_copy(x_hbm.at[b, pl.ds(off, CW)], x_v.at[b])

            def grp(g, c2):
                go = g * L
                pv = p_v[pl.ds(go, L)]
                for b in range(B):
                    x_v[b, pl.ds(go, L)] = x_v[b, pl.ds(go, L)] + pv
                return c2

            lax.fori_loop(0, CW // L, grp, 0)
            for b in range(B):
                pltpu.sync_copy(x_v.at[b], o_hbm.at[b, pl.ds(off, CW)])
            return carry

        lax.fori_loop(0, n_chunks, chunk, 0)

    out = k(x, p)
    return out.reshape(B, S, D)


# speedup vs baseline: 1.3842x; 1.3842x over previous
import functools

import jax
import jax.numpy as jnp
from jax import lax
from jax.experimental import pallas as pl
from jax.experimental.pallas import tpu as pltpu
from jax.experimental.pallas import tpu_sc as plsc


def kernel(inputs, pos_embedding):
    B, S, D = inputs.shape
    x = inputs.reshape(B, S * D)
    p = pos_embedding.reshape(S * D)

    info = plsc.get_sparse_core_info()
    NC, NS, L = info.num_cores, info.num_subcores, info.num_lanes
    NW = NC * NS
    rows_w = S // NW
    CH = 8
    NCH = rows_w // CH
    CW = CH * D
    NB = 3

    mesh = plsc.VectorSubcoreMesh(core_axis_name="c", subcore_axis_name="s")

    @functools.partial(
        pl.kernel,
        mesh=mesh,
        out_type=jax.ShapeDtypeStruct((B, S * D), jnp.float32),
        scratch_types=(
            [pltpu.VMEM((CW,), jnp.float32)] * NB
            + [pltpu.VMEM((B, CW), jnp.float32)] * NB
            + [pltpu.SemaphoreType.DMA] * (2 * NB)
        ),
    )
    def k(x_hbm, p_hbm, o_hbm, *scr):
        p_bufs = scr[:NB]
        x_bufs = scr[NB : 2 * NB]
        in_sems = scr[2 * NB : 3 * NB]
        out_sems = scr[3 * NB :]
        wid = lax.axis_index("s") * NC + lax.axis_index("c")
        base = wid * (rows_w * D)

        def in_copies(ci):
            s = ci % NB
            off = base + ci * CW
            cps = [pltpu.make_async_copy(p_hbm.at[pl.ds(off, CW)], p_bufs[s], in_sems[s])]
            for b in range(B):
                cps.append(
                    pltpu.make_async_copy(
                        x_hbm.at[b, pl.ds(off, CW)], x_bufs[s].at[b], in_sems[s]
                    )
                )
            return cps

        def out_copies(ci):
            s = ci % NB
            off = base + ci * CW
            return [
                pltpu.make_async_copy(
                    x_bufs[s].at[b], o_hbm.at[b, pl.ds(off, CW)], out_sems[s]
                )
                for b in range(B)
            ]

        for ci in range(min(2, NCH)):
            for c in in_copies(ci):
                c.start()

        for ci in range(NCH):
            s = ci % NB
            for c in in_copies(ci):
                c.wait()

            pb = p_bufs[s]
            xb = x_bufs[s]

            @plsc.parallel_loop(0, CW // L, unroll=8)
            def _grp(g):
                go = g * L
                pv = pb[pl.ds(go, L)]
                for b in range(B):
                    xb[b, pl.ds(go, L)] = xb[b, pl.ds(go, L)] + pv

            for c in out_copies(ci):
                c.start()
            if ci + 2 < NCH:
                if ci - 1 >= 0:
                    for c in out_copies(ci - 1):
                        c.wait()
                for c in in_copies(ci + 2):
                    c.start()

        for ci in range(max(0, NCH - 3), NCH):
            for c in out_copies(ci):
                c.wait()

    out = k(x, p)
    return out.reshape(B, S, D)
